# Initial kernel scaffold; baseline (speedup 1.0000x reference)
import jax
import jax.numpy as jnp
from jax.experimental import pallas as pl


def kernel(xyz, params):
    raise NotImplementedError("write your pallas kernel here")


# speedup vs baseline: 4.6248x; 4.6248x over previous
import functools

import jax
import jax.numpy as jnp
from jax import lax
from jax.experimental import pallas as pl

F32 = jnp.float32
HIGH = lax.Precision.HIGHEST


def _twosum(s, c, v):
    t = s + v
    e = jnp.where(jnp.abs(s) >= jnp.abs(v), (s - t) + v, (v - t) + s)
    return t, c + e


def _fps_kernel(xs_ref, ys_ref, zs_ref, nx_ref, ny_ref, nz_ref, *, npoint):
    X = xs_ref[...]
    Y = ys_ref[...]
    Z = zs_ref[...]
    B, N = X.shape
    iota_n = lax.broadcasted_iota(jnp.int32, (B, N), 1)
    iota_s = lax.broadcasted_iota(jnp.int32, (B, npoint), 1)

    def body(i, state):
        dist, far, nx, ny, nz = state
        oh = (iota_n == far).astype(F32)
        cx = jnp.sum(X * oh, axis=1, keepdims=True)
        cy = jnp.sum(Y * oh, axis=1, keepdims=True)
        cz = jnp.sum(Z * oh, axis=1, keepdims=True)
        nx = jnp.where(iota_s == i, cx, nx)
        ny = jnp.where(iota_s == i, cy, ny)
        nz = jnp.where(iota_s == i, cz, nz)
        d = (X - cx) ** 2 + (Y - cy) ** 2 + (Z - cz) ** 2
        dist = jnp.minimum(dist, d)
        m = jnp.max(dist, axis=1, keepdims=True)
        far = jnp.min(jnp.where(dist == m, iota_n, N), axis=1, keepdims=True)
        return dist, far, nx, ny, nz

    dist0 = jnp.full((B, N), 1e10, F32)
    far0 = jnp.zeros((B, 1), jnp.int32)
    z0 = jnp.zeros((B, npoint), F32)
    _, _, nx, ny, nz = lax.fori_loop(0, npoint, body, (dist0, far0, z0, z0, z0))
    nx_ref[...] = nx
    ny_ref[...] = ny
    nz_ref[...] = nz


def _fps(xyz, npoint):
    B, N, _ = xyz.shape
    xs, ys, zs = xyz[:, :, 0], xyz[:, :, 1], xyz[:, :, 2]
    out = jax.ShapeDtypeStruct((B, npoint), F32)
    nx, ny, nz = pl.pallas_call(
        functools.partial(_fps_kernel, npoint=npoint),
        grid=(1,),
        in_specs=[pl.BlockSpec((B, N), lambda i: (0, 0))] * 3,
        out_specs=[pl.BlockSpec((B, npoint), lambda i: (0, 0))] * 3,
        out_shape=[out, out, out],
    )(xs, ys, zs)
    return jnp.stack([nx, ny, nz], axis=2)


def _bq_kernel(q_ref, pt_ref, idx_ref, *, radius2, K):
    q = q_ref[0]
    pt = pt_ref[0]
    S = q.shape[0]
    N = pt.shape[1]
    qn = jnp.sum(q * q, axis=1, keepdims=True)
    pn = jnp.sum(pt * pt, axis=0, keepdims=True)
    cross = lax.dot_general(q, pt, (((1,), (0,)), ((), ())))
    d = qn + pn - 2.0 * cross
    iota_n = lax.broadcasted_iota(jnp.int32, (S, N), 1)
    big = jnp.int32(N)
    key = jnp.where(d <= radius2, iota_n, big)
    iota_k = lax.broadcasted_iota(jnp.int32, (S, K), 1)
    acc = jnp.zeros((S, K), jnp.int32)
    first = None
    for j in range(K):
        mj = jnp.min(key, axis=1, keepdims=True)
        key = jnp.where(key == mj, big, key)
        if j == 0:
            first = mj
            mjf = mj
        else:
            mjf = jnp.where(mj >= big, first, mj)
        acc = jnp.where(iota_k == j, mjf, acc)
    idx_ref[0] = acc


def _ball_query(new_xyz, xyz, radius, K):
    B, S, _ = new_xyz.shape
    N = xyz.shape[1]
    xyz_t = jnp.transpose(xyz, (0, 2, 1))
    return pl.pallas_call(
        functools.partial(_bq_kernel, radius2=radius * radius, K=K),
        grid=(B,),
        in_specs=[
            pl.BlockSpec((1, S, 3), lambda b: (b, 0, 0)),
            pl.BlockSpec((1, 3, N), lambda b: (b, 0, 0)),
        ],
        out_specs=pl.BlockSpec((1, S, K), lambda b: (b, 0, 0)),
        out_shape=jax.ShapeDtypeStruct((B, S, K), jnp.int32),
    )(new_xyz, xyz_t)


def _gmlp1_kernel(pts_ref, idx_ref, qf_ref, w_ref, b_ref,
                  z_ref, s1_ref, c1_ref, *, K, KB):
    pts = pts_ref[0]
    qf = qf_ref[0]
    idxv = idx_ref[0]
    S, C = qf.shape
    N = pts.shape[0]
    w = w_ref[...]
    b = b_ref[...]
    kb = pl.program_id(1)
    iota_n = lax.broadcasted_iota(jnp.int32, (S, N), 1)
    iota_k = lax.broadcasted_iota(jnp.int32, (S, K), 1)
    @pl.when(jnp.logical_and(pl.program_id(0) == 0, kb == 0))
    def _():
        s1_ref[...] = jnp.zeros_like(s1_ref)
        c1_ref[...] = jnp.zeros_like(c1_ref)

    s1 = s1_ref[...]
    c1 = c1_ref[...]
    for k in range(KB):
        col = kb * KB + k
        ik = jnp.sum(jnp.where(iota_k == col, idxv, 0), axis=1, keepdims=True)
        oh = (ik == iota_n).astype(F32)
        g = lax.dot_general(oh, pts, (((1,), (0,)), ((), ())), precision=HIGH)
        gn = g - qf
        zk = lax.dot_general(gn, w, (((1,), (1,)), ((), ()))) + b
        z_ref[k] = zk
        s1, c1 = _twosum(s1, c1, jnp.sum(zk, axis=0, keepdims=True))

    s1_ref[...] = s1
    c1_ref[...] = c1


def _gmlp1(ptsfull, idx, qfull, w, b, K, KB):
    B, N, C = ptsfull.shape
    S = idx.shape[1]
    O = w.shape[0]
    nkb = K // KB
    z, s1, c1 = pl.pallas_call(
        functools.partial(_gmlp1_kernel, K=K, KB=KB),
        grid=(B, nkb),
        in_specs=[
            pl.BlockSpec((1, N, C), lambda b_, kb: (b_, 0, 0)),
            pl.BlockSpec((1, S, K), lambda b_, kb: (b_, 0, 0)),
            pl.BlockSpec((1, S, C), lambda b_, kb: (b_, 0, 0)),
            pl.BlockSpec((O, C), lambda b_, kb: (0, 0)),
            pl.BlockSpec((1, O), lambda b_, kb: (0, 0)),
        ],
        out_specs=[
            pl.BlockSpec((KB, S, O), lambda b_, kb: (b_ * nkb + kb, 0, 0)),
            pl.BlockSpec((1, O), lambda b_, kb: (0, 0)),
            pl.BlockSpec((1, O), lambda b_, kb: (0, 0)),
        ],
        out_shape=[
            jax.ShapeDtypeStruct((B * K, S, O), F32),
            jax.ShapeDtypeStruct((1, O), F32),
            jax.ShapeDtypeStruct((1, O), F32),
        ],
    )(ptsfull, idx, qfull, w, b.reshape(1, O))
    return z, s1 + c1


def _mlp_kernel(z_ref, mean_ref, sq_ref, ga_ref, be_ref, w_ref, b_ref,
                o_ref, s1_ref, c1_ref, *, KB):
    mean = mean_ref[...]
    sq = sq_ref[...]
    ga = ga_ref[...]
    be = be_ref[...]
    w = w_ref[...]
    b = b_ref[...]
    @pl.when(pl.program_id(0) == 0)
    def _():
        s1_ref[...] = jnp.zeros_like(s1_ref)
        c1_ref[...] = jnp.zeros_like(c1_ref)

    s1 = s1_ref[...]
    c1 = c1_ref[...]
    for k in range(KB):
        xn = (z_ref[k] - mean) / sq
        x = jnp.maximum(ga * xn + be, 0.0)
        zk = lax.dot_general(x, w, (((1,), (1,)), ((), ()))) + b
        o_ref[k] = zk
        s1, c1 = _twosum(s1, c1, jnp.sum(zk, axis=0, keepdims=True))

    s1_ref[...] = s1
    c1_ref[...] = c1


def _mlp(z, S, mean, sq, ga, be, w, b, KB):
    BK = z.shape[0]
    Oin = z.shape[2]
    O = w.shape[0]
    steps = BK // KB
    out, s1, c1 = pl.pallas_call(
        functools.partial(_mlp_kernel, KB=KB),
        grid=(steps,),
        in_specs=[
            pl.BlockSpec((KB, S, Oin), lambda i: (i, 0, 0)),
            pl.BlockSpec((1, Oin), lambda i: (0, 0)),
            pl.BlockSpec((1, Oin), lambda i: (0, 0)),
            pl.BlockSpec((1, Oin), lambda i: (0, 0)),
            pl.BlockSpec((1, Oin), lambda i: (0, 0)),
            pl.BlockSpec((O, Oin), lambda i: (0, 0)),
            pl.BlockSpec((1, O), lambda i: (0, 0)),
        ],
        out_specs=[
            pl.BlockSpec((KB, S, O), lambda i: (i, 0, 0)),
            pl.BlockSpec((1, O), lambda i: (0, 0)),
            pl.BlockSpec((1, O), lambda i: (0, 0)),
        ],
        out_shape=[
            jax.ShapeDtypeStruct((BK, S, O), F32),
            jax.ShapeDtypeStruct((1, O), F32),
            jax.ShapeDtypeStruct((1, O), F32),
        ],
    )(z, mean.reshape(1, Oin), sq.reshape(1, Oin), ga.reshape(1, Oin),
      be.reshape(1, Oin), w, b.reshape(1, O))
    return out, s1 + c1


def _maxpool_kernel(z_ref, mean_ref, sq_ref, ga_ref, be_ref, o_ref, *, KB):
    mean = mean_ref[...]
    sq = sq_ref[...]
    ga = ga_ref[...]
    be = be_ref[...]
    xn = (z_ref[0] - mean) / sq
    m = jnp.maximum(ga * xn + be, 0.0)
    for k in range(1, KB):
        xn = (z_ref[k] - mean) / sq
        m = jnp.maximum(m, jnp.maximum(ga * xn + be, 0.0))

    kb = pl.program_id(1)

    @pl.when(kb == 0)
    def _():
        o_ref[0] = m

    @pl.when(kb != 0)
    def _():
        o_ref[0] = jnp.maximum(o_ref[0], m)


def _maxpool(z, B, S, mean, sq, ga, be, KB):
    BK = z.shape[0]
    O = z.shape[2]
    K = BK // B
    nkb = K // KB
    return pl.pallas_call(
        functools.partial(_maxpool_kernel, KB=KB),
        grid=(B, nkb),
        in_specs=[
            pl.BlockSpec((KB, S, O), lambda b_, kb: (b_ * nkb + kb, 0, 0)),
            pl.BlockSpec((1, O), lambda b_, kb: (0, 0)),
            pl.BlockSpec((1, O), lambda b_, kb: (0, 0)),
            pl.BlockSpec((1, O), lambda b_, kb: (0, 0)),
            pl.BlockSpec((1, O), lambda b_, kb: (0, 0)),
        ],
        out_specs=pl.BlockSpec((1, S, O), lambda b_, kb: (b_, 0, 0)),
        out_shape=jax.ShapeDtypeStruct((B, S, O), F32),
    )(z, mean.reshape(1, O), sq.reshape(1, O), ga.reshape(1, O),
      be.reshape(1, O))


def _lin_kernel(x_ref, w_ref, b_ref, o_ref, s1_ref, c1_ref):
    x = x_ref[0]
    b = b_ref[...]
    z = lax.dot_general(x, w_ref[...], (((1,), (1,)), ((), ()))) + b
    o_ref[0] = z

    @pl.when(pl.program_id(0) == 0)
    def _():
        s1_ref[...] = jnp.zeros_like(s1_ref)
        c1_ref[...] = jnp.zeros_like(c1_ref)

    s1, c1 = _twosum(s1_ref[...], c1_ref[...],
                     jnp.sum(z, axis=0, keepdims=True))
    s1_ref[...] = s1
    c1_ref[...] = c1


def _lin(x, w, b):
    B, S, C = x.shape
    O = w.shape[0]
    z, s1, c1 = pl.pallas_call(
        _lin_kernel,
        grid=(B,),
        in_specs=[
            pl.BlockSpec((1, S, C), lambda b_: (b_, 0, 0)),
            pl.BlockSpec((O, C), lambda b_: (0, 0)),
            pl.BlockSpec((1, O), lambda b_: (0, 0)),
        ],
        out_specs=[
            pl.BlockSpec((1, S, O), lambda b_: (b_, 0, 0)),
            pl.BlockSpec((1, O), lambda b_: (0, 0)),
            pl.BlockSpec((1, O), lambda b_: (0, 0)),
        ],
        out_shape=[
            jax.ShapeDtypeStruct((B, S, O), F32),
            jax.ShapeDtypeStruct((1, O), F32),
            jax.ShapeDtypeStruct((1, O), F32),
        ],
    )(x, w, b.reshape(1, O))
    return z, s1 + c1


def _cvar_kernel(z_ref, mean_ref, s2_ref, c2_ref, *, KB):
    mean = mean_ref[...]

    @pl.when(pl.program_id(0) == 0)
    def _():
        s2_ref[...] = jnp.zeros_like(s2_ref)
        c2_ref[...] = jnp.zeros_like(c2_ref)

    s2 = s2_ref[...]
    c2 = c2_ref[...]
    for k in range(KB):
        d = z_ref[k] - mean
        s2, c2 = _twosum(s2, c2, jnp.sum(d * d, axis=0, keepdims=True))

    s2_ref[...] = s2
    c2_ref[...] = c2


def _cvar(z, S, mean, KB):
    BK = z.shape[0]
    O = z.shape[2]
    steps = BK // KB
    s2, c2 = pl.pallas_call(
        functools.partial(_cvar_kernel, KB=KB),
        grid=(steps,),
        in_specs=[
            pl.BlockSpec((KB, S, O), lambda i: (i, 0, 0)),
            pl.BlockSpec((1, O), lambda i: (0, 0)),
        ],
        out_specs=[
            pl.BlockSpec((1, O), lambda i: (0, 0)),
            pl.BlockSpec((1, O), lambda i: (0, 0)),
        ],
        out_shape=[
            jax.ShapeDtypeStruct((1, O), F32),
            jax.ShapeDtypeStruct((1, O), F32),
        ],
    )(z, mean.reshape(1, O))
    return s2 + c2


def _gmax_kernel(z_ref, mean_ref, sq_ref, ga_ref, be_ref, o_ref):
    xn = (z_ref[0] - mean_ref[...]) / sq_ref[...]
    v = jnp.maximum(ga_ref[...] * xn + be_ref[...], 0.0)
    o_ref[0] = jnp.max(v, axis=0, keepdims=True)


def _gmax(z, mean, sq, ga, be):
    B, S, O = z.shape
    out = pl.pallas_call(
        _gmax_kernel,
        grid=(B,),
        in_specs=[
            pl.BlockSpec((1, S, O), lambda b_: (b_, 0, 0)),
            pl.BlockSpec((1, O), lambda b_: (0, 0)),
            pl.BlockSpec((1, O), lambda b_: (0, 0)),
            pl.BlockSpec((1, O), lambda b_: (0, 0)),
            pl.BlockSpec((1, O), lambda b_: (0, 0)),
        ],
        out_specs=pl.BlockSpec((1, 1, O), lambda b_: (b_, 0, 0)),
        out_shape=jax.ShapeDtypeStruct((B, 1, O), F32),
    )(z, mean.reshape(1, O), sq.reshape(1, O), ga.reshape(1, O),
      be.reshape(1, O))
    return out.reshape(B, O)


def _bn_stats(z, s1, S, count, KB):
    mean = (s1 / count).reshape(-1)
    s2c = _cvar(z, S, mean, KB)
    sq = jnp.sqrt(s2c.reshape(-1) / count + 1e-5)
    return mean, sq


def _kb_for(K, S, O):
    kb = K
    while kb > 1 and kb * S * O * 4 > 4 * 1024 * 1024:
        kb //= 2
    return kb


def _sa_layer(xyz, points, layer_params, npoint, radius, K):
    B, N, _ = xyz.shape
    new_xyz = _fps(xyz, npoint)
    idx = _ball_query(new_xyz, xyz, radius, K)
    if points is None:
        ptsfull = xyz
    else:
        ptsfull = jnp.concatenate([xyz, points], axis=2)
    C = ptsfull.shape[2]
    S = npoint
    qfull = jnp.concatenate(
        [new_xyz, jnp.zeros((B, S, C - 3), F32)], axis=2)

    (w1, b1, g1, be1), (w2, b2, g2, be2), (w3, b3, g3, be3) = layer_params
    count = float(B * K * S)
    kb1 = _kb_for(K, S, w1.shape[0])
    z, s1 = _gmlp1(ptsfull, idx, qfull, w1, b1, K, kb1)
    mean, sq = _bn_stats(z, s1, S, count, kb1)
    kb2 = _kb_for(K, S, w2.shape[0])
    z, s1 = _mlp(z, S, mean, sq, g1, be1, w2, b2, kb2)
    mean, sq = _bn_stats(z, s1, S, count, kb2)
    kb3 = _kb_for(K, S, w3.shape[0])
    z, s1 = _mlp(z, S, mean, sq, g2, be2, w3, b3, kb3)
    mean, sq = _bn_stats(z, s1, S, count, kb3)
    new_points = _maxpool(z, B, S, mean, sq, g3, be3, kb3)
    return new_xyz, new_points


def _sa_group_all(xyz, points, layer_params):
    B, N, _ = xyz.shape
    ptsfull = jnp.concatenate([xyz, points], axis=2)
    (w1, b1, g1, be1), (w2, b2, g2, be2), (w3, b3, g3, be3) = layer_params
    count = float(B * N)
    z, s1 = _lin(ptsfull, w1, b1)
    mean, sq = _bn_stats(z, s1, N, count, 1)
    z, s1 = _mlp(z, N, mean, sq, g1, be1, w2, b2, 1)
    mean, sq = _bn_stats(z, s1, N, count, 1)
    z, s1 = _mlp(z, N, mean, sq, g2, be2, w3, b3, 1)
    mean, sq = _bn_stats(z, s1, N, count, 1)
    return _gmax(z, mean, sq, g3, be3)


def kernel(xyz, params):
    B = xyz.shape[0]
    pts = jnp.transpose(xyz, (0, 2, 1))
    xyz3 = pts[:, :, :3]
    norm = pts[:, :, 3:]
    l1_xyz, l1_points = _sa_layer(xyz3, norm, params['sa1'], 778, 0.2, 16)
    l2_xyz, l2_points = _sa_layer(l1_xyz, l1_points, params['sa2'], 388, 0.4, 32)
    l3_xyz, l3_points = _sa_layer(l2_xyz, l2_points, params['sa3'], 194, 0.8, 64)
    out = _sa_group_all(l3_xyz, l3_points, params['sa4'])
    return out.reshape(B, 512)


# speedup vs baseline: 9.4757x; 2.0489x over previous
import functools

import jax
import jax.numpy as jnp
from jax import lax
from jax.experimental import pallas as pl
from jax.experimental.pallas import tpu as pltpu
from jax.experimental.pallas import tpu_sc as plsc

F32 = jnp.float32
HIGH = lax.Precision.HIGHEST


def _twosum(s, c, v):
    t = s + v
    e = jnp.where(jnp.abs(s) >= jnp.abs(v), (s - t) + v, (v - t) + s)
    return t, c + e


def _fps_kernel(xs_ref, ys_ref, zs_ref, nx_ref, ny_ref, nz_ref, *, npoint):
    X = xs_ref[...]
    Y = ys_ref[...]
    Z = zs_ref[...]
    B, N = X.shape
    iota_n = lax.broadcasted_iota(jnp.int32, (B, N), 1)
    iota_s = lax.broadcasted_iota(jnp.int32, (B, npoint), 1)

    def body(i, state):
        dist, far, nx, ny, nz = state
        oh = (iota_n == far).astype(F32)
        cx = jnp.sum(X * oh, axis=1, keepdims=True)
        cy = jnp.sum(Y * oh, axis=1, keepdims=True)
        cz = jnp.sum(Z * oh, axis=1, keepdims=True)
        nx = jnp.where(iota_s == i, cx, nx)
        ny = jnp.where(iota_s == i, cy, ny)
        nz = jnp.where(iota_s == i, cz, nz)
        d = (X - cx) ** 2 + (Y - cy) ** 2 + (Z - cz) ** 2
        dist = jnp.minimum(dist, d)
        m = jnp.max(dist, axis=1, keepdims=True)
        far = jnp.min(jnp.where(dist == m, iota_n, N), axis=1, keepdims=True)
        return dist, far, nx, ny, nz

    dist0 = jnp.full((B, N), 1e10, F32)
    far0 = jnp.zeros((B, 1), jnp.int32)
    z0 = jnp.zeros((B, npoint), F32)
    _, _, nx, ny, nz = lax.fori_loop(0, npoint, body, (dist0, far0, z0, z0, z0))
    nx_ref[...] = nx
    ny_ref[...] = ny
    nz_ref[...] = nz


def _fps(xyz, npoint):
    B, N, _ = xyz.shape
    xs, ys, zs = xyz[:, :, 0], xyz[:, :, 1], xyz[:, :, 2]
    out = jax.ShapeDtypeStruct((B, npoint), F32)
    nx, ny, nz = pl.pallas_call(
        functools.partial(_fps_kernel, npoint=npoint),
        grid=(1,),
        in_specs=[pl.BlockSpec((B, N), lambda i: (0, 0))] * 3,
        out_specs=[pl.BlockSpec((B, npoint), lambda i: (0, 0))] * 3,
        out_shape=[out, out, out],
    )(xs, ys, zs)
    return jnp.stack([nx, ny, nz], axis=2)


def _bq_kernel(q_ref, pt_ref, idx_ref, *, radius2, K):
    q = q_ref[0]
    pt = pt_ref[0]
    S = q.shape[0]
    N = pt.shape[1]
    qn = jnp.sum(q * q, axis=1, keepdims=True)
    pn = jnp.sum(pt * pt, axis=0, keepdims=True)
    cross = lax.dot_general(q, pt, (((1,), (0,)), ((), ())))
    d = qn + pn - 2.0 * cross
    iota_n = lax.broadcasted_iota(jnp.int32, (S, N), 1)
    big = jnp.int32(N)
    key = jnp.where(d <= radius2, iota_n, big)
    iota_k = lax.broadcasted_iota(jnp.int32, (S, K), 1)
    acc = jnp.zeros((S, K), jnp.int32)
    first = None
    for j in range(K):
        mj = jnp.min(key, axis=1, keepdims=True)
        key = jnp.where(key == mj, big, key)
        if j == 0:
            first = mj
            mjf = mj
        else:
            mjf = jnp.where(mj >= big, first, mj)
        acc = jnp.where(iota_k == j, mjf, acc)
    idx_ref[0] = acc


def _ball_query(new_xyz, xyz, radius, K):
    B, S, _ = new_xyz.shape
    N = xyz.shape[1]
    xyz_t = jnp.transpose(xyz, (0, 2, 1))
    return pl.pallas_call(
        functools.partial(_bq_kernel, radius2=radius * radius, K=K),
        grid=(B,),
        in_specs=[
            pl.BlockSpec((1, S, 3), lambda b: (b, 0, 0)),
            pl.BlockSpec((1, 3, N), lambda b: (b, 0, 0)),
        ],
        out_specs=pl.BlockSpec((1, S, K), lambda b: (b, 0, 0)),
        out_shape=jax.ShapeDtypeStruct((B, S, K), jnp.int32),
    )(new_xyz, xyz_t)


def _sc_gather(table, idx_flat):
    R = idx_flat.shape[0]
    Cp = table.shape[1]
    info = plsc.get_sparse_core_info()
    NW = info.num_cores * info.num_subcores
    CH = 256
    n_chunks = R // CH
    rounds = -(-n_chunks // NW)
    mesh = plsc.VectorSubcoreMesh(core_axis_name="c", subcore_axis_name="s")

    @functools.partial(
        pl.kernel, mesh=mesh,
        out_type=jax.ShapeDtypeStruct((R, Cp), F32),
        scratch_types=[
            pltpu.VMEM((CH,), jnp.int32),
            pltpu.VMEM((CH, Cp), F32),
            pltpu.SemaphoreType.DMA,
        ],
    )
    def k(table_hbm, idx_hbm, out_hbm, idx_v, rows_v, sem):
        wid = lax.axis_index("s") * info.num_cores + lax.axis_index("c")
        for i in range(rounds):
            ck = wid + i * NW

            @pl.when(ck < n_chunks)
            def _():
                base = ck * CH
                pltpu.sync_copy(idx_hbm.at[pl.ds(base, CH)], idx_v)
                pltpu.async_copy(table_hbm.at[idx_v], rows_v, sem).wait()
                pltpu.sync_copy(rows_v, out_hbm.at[pl.ds(base, CH)])

    return k(table, idx_flat)


def _gmlp1_kernel(g_ref, qf_ref, w_ref, b_ref,
                  z_ref, s1_ref, c1_ref, *, KB):
    qf = qf_ref[0]
    w = w_ref[...]
    b = b_ref[...]
    kb = pl.program_id(1)

    @pl.when(jnp.logical_and(pl.program_id(0) == 0, kb == 0))
    def _():
        s1_ref[...] = jnp.zeros_like(s1_ref)
        c1_ref[...] = jnp.zeros_like(c1_ref)

    s1 = s1_ref[...]
    c1 = c1_ref[...]
    for k in range(KB):
        gn = g_ref[k] - qf
        zk = lax.dot_general(gn, w, (((1,), (1,)), ((), ()))) + b
        z_ref[k] = zk
        s1, c1 = _twosum(s1, c1, jnp.sum(zk, axis=0, keepdims=True))

    s1_ref[...] = s1
    c1_ref[...] = c1


def _gmlp1(ptsfull, idx, qfull, w, b, K, KB):
    B, N, C = ptsfull.shape
    S = idx.shape[1]
    O = w.shape[0]
    Cp = -(-C // 128) * 128
    pad = Cp - C
    table = jnp.concatenate(
        [ptsfull, jnp.zeros((B, N, pad), F32)], axis=2).reshape(B * N, Cp)
    qfp = jnp.concatenate([qfull, jnp.zeros((B, S, pad), F32)], axis=2)
    wp = jnp.concatenate([w, jnp.zeros((O, pad), F32)], axis=1)
    offs = (jnp.arange(B, dtype=jnp.int32) * N)[:, None, None]
    idx_flat = (jnp.transpose(idx, (0, 2, 1)) + offs).reshape(-1)
    g = _sc_gather(table, idx_flat).reshape(B * K, S, Cp)

    nkb = K // KB
    z, s1, c1 = pl.pallas_call(
        functools.partial(_gmlp1_kernel, KB=KB),
        grid=(B, nkb),
        in_specs=[
            pl.BlockSpec((KB, S, Cp), lambda b_, kb: (b_ * nkb + kb, 0, 0)),
            pl.BlockSpec((1, S, Cp), lambda b_, kb: (b_, 0, 0)),
            pl.BlockSpec((O, Cp), lambda b_, kb: (0, 0)),
            pl.BlockSpec((1, O), lambda b_, kb: (0, 0)),
        ],
        out_specs=[
            pl.BlockSpec((KB, S, O), lambda b_, kb: (b_ * nkb + kb, 0, 0)),
            pl.BlockSpec((1, O), lambda b_, kb: (0, 0)),
            pl.BlockSpec((1, O), lambda b_, kb: (0, 0)),
        ],
        out_shape=[
            jax.ShapeDtypeStruct((B * K, S, O), F32),
            jax.ShapeDtypeStruct((1, O), F32),
            jax.ShapeDtypeStruct((1, O), F32),
        ],
    )(g, qfp, wp, b.reshape(1, O))
    return z, s1 + c1


def _mlp_kernel(z_ref, mean_ref, sq_ref, ga_ref, be_ref, w_ref, b_ref,
                o_ref, s1_ref, c1_ref, *, KB):
    mean = mean_ref[...]
    sq = sq_ref[...]
    ga = ga_ref[...]
    be = be_ref[...]
    w = w_ref[...]
    b = b_ref[...]
    @pl.when(pl.program_id(0) == 0)
    def _():
        s1_ref[...] = jnp.zeros_like(s1_ref)
        c1_ref[...] = jnp.zeros_like(c1_ref)

    s1 = s1_ref[...]
    c1 = c1_ref[...]
    for k in range(KB):
        xn = (z_ref[k] - mean) / sq
        x = jnp.maximum(ga * xn + be, 0.0)
        zk = lax.dot_general(x, w, (((1,), (1,)), ((), ()))) + b
        o_ref[k] = zk
        s1, c1 = _twosum(s1, c1, jnp.sum(zk, axis=0, keepdims=True))

    s1_ref[...] = s1
    c1_ref[...] = c1


def _mlp(z, S, mean, sq, ga, be, w, b, KB):
    BK = z.shape[0]
    Oin = z.shape[2]
    O = w.shape[0]
    steps = BK // KB
    out, s1, c1 = pl.pallas_call(
        functools.partial(_mlp_kernel, KB=KB),
        grid=(steps,),
        in_specs=[
            pl.BlockSpec((KB, S, Oin), lambda i: (i, 0, 0)),
            pl.BlockSpec((1, Oin), lambda i: (0, 0)),
            pl.BlockSpec((1, Oin), lambda i: (0, 0)),
            pl.BlockSpec((1, Oin), lambda i: (0, 0)),
            pl.BlockSpec((1, Oin), lambda i: (0, 0)),
            pl.BlockSpec((O, Oin), lambda i: (0, 0)),
            pl.BlockSpec((1, O), lambda i: (0, 0)),
        ],
        out_specs=[
            pl.BlockSpec((KB, S, O), lambda i: (i, 0, 0)),
            pl.BlockSpec((1, O), lambda i: (0, 0)),
            pl.BlockSpec((1, O), lambda i: (0, 0)),
        ],
        out_shape=[
            jax.ShapeDtypeStruct((BK, S, O), F32),
            jax.ShapeDtypeStruct((1, O), F32),
            jax.ShapeDtypeStruct((1, O), F32),
        ],
    )(z, mean.reshape(1, Oin), sq.reshape(1, Oin), ga.reshape(1, Oin),
      be.reshape(1, Oin), w, b.reshape(1, O))
    return out, s1 + c1


def _maxpool_kernel(z_ref, mean_ref, sq_ref, ga_ref, be_ref, o_ref, *, KB):
    mean = mean_ref[...]
    sq = sq_ref[...]
    ga = ga_ref[...]
    be = be_ref[...]
    xn = (z_ref[0] - mean) / sq
    m = jnp.maximum(ga * xn + be, 0.0)
    for k in range(1, KB):
        xn = (z_ref[k] - mean) / sq
        m = jnp.maximum(m, jnp.maximum(ga * xn + be, 0.0))

    kb = pl.program_id(1)

    @pl.when(kb == 0)
    def _():
        o_ref[0] = m

    @pl.when(kb != 0)
    def _():
        o_ref[0] = jnp.maximum(o_ref[0], m)


def _maxpool(z, B, S, mean, sq, ga, be, KB):
    BK = z.shape[0]
    O = z.shape[2]
    K = BK // B
    nkb = K // KB
    return pl.pallas_call(
        functools.partial(_maxpool_kernel, KB=KB),
        grid=(B, nkb),
        in_specs=[
            pl.BlockSpec((KB, S, O), lambda b_, kb: (b_ * nkb + kb, 0, 0)),
            pl.BlockSpec((1, O), lambda b_, kb: (0, 0)),
            pl.BlockSpec((1, O), lambda b_, kb: (0, 0)),
            pl.BlockSpec((1, O), lambda b_, kb: (0, 0)),
            pl.BlockSpec((1, O), lambda b_, kb: (0, 0)),
        ],
        out_specs=pl.BlockSpec((1, S, O), lambda b_, kb: (b_, 0, 0)),
        out_shape=jax.ShapeDtypeStruct((B, S, O), F32),
    )(z, mean.reshape(1, O), sq.reshape(1, O), ga.reshape(1, O),
      be.reshape(1, O))


def _lin_kernel(x_ref, w_ref, b_ref, o_ref, s1_ref, c1_ref):
    x = x_ref[0]
    b = b_ref[...]
    z = lax.dot_general(x, w_ref[...], (((1,), (1,)), ((), ()))) + b
    o_ref[0] = z

    @pl.when(pl.program_id(0) == 0)
    def _():
        s1_ref[...] = jnp.zeros_like(s1_ref)
        c1_ref[...] = jnp.zeros_like(c1_ref)

    s1, c1 = _twosum(s1_ref[...], c1_ref[...],
                     jnp.sum(z, axis=0, keepdims=True))
    s1_ref[...] = s1
    c1_ref[...] = c1


def _lin(x, w, b):
    B, S, C = x.shape
    O = w.shape[0]
    z, s1, c1 = pl.pallas_call(
        _lin_kernel,
        grid=(B,),
        in_specs=[
            pl.BlockSpec((1, S, C), lambda b_: (b_, 0, 0)),
            pl.BlockSpec((O, C), lambda b_: (0, 0)),
            pl.BlockSpec((1, O), lambda b_: (0, 0)),
        ],
        out_specs=[
            pl.BlockSpec((1, S, O), lambda b_: (b_, 0, 0)),
            pl.BlockSpec((1, O), lambda b_: (0, 0)),
            pl.BlockSpec((1, O), lambda b_: (0, 0)),
        ],
        out_shape=[
            jax.ShapeDtypeStruct((B, S, O), F32),
            jax.ShapeDtypeStruct((1, O), F32),
            jax.ShapeDtypeStruct((1, O), F32),
        ],
    )(x, w, b.reshape(1, O))
    return z, s1 + c1


def _cvar_kernel(z_ref, mean_ref, s2_ref, c2_ref, *, KB):
    mean = mean_ref[...]

    @pl.when(pl.program_id(0) == 0)
    def _():
        s2_ref[...] = jnp.zeros_like(s2_ref)
        c2_ref[...] = jnp.zeros_like(c2_ref)

    s2 = s2_ref[...]
    c2 = c2_ref[...]
    for k in range(KB):
        d = z_ref[k] - mean
        s2, c2 = _twosum(s2, c2, jnp.sum(d * d, axis=0, keepdims=True))

    s2_ref[...] = s2
    c2_ref[...] = c2


def _cvar(z, S, mean, KB):
    BK = z.shape[0]
    O = z.shape[2]
    steps = BK // KB
    s2, c2 = pl.pallas_call(
        functools.partial(_cvar_kernel, KB=KB),
        grid=(steps,),
        in_specs=[
            pl.BlockSpec((KB, S, O), lambda i: (i, 0, 0)),
            pl.BlockSpec((1, O), lambda i: (0, 0)),
        ],
        out_specs=[
            pl.BlockSpec((1, O), lambda i: (0, 0)),
            pl.BlockSpec((1, O), lambda i: (0, 0)),
        ],
        out_shape=[
            jax.ShapeDtypeStruct((1, O), F32),
            jax.ShapeDtypeStruct((1, O), F32),
        ],
    )(z, mean.reshape(1, O))
    return s2 + c2


def _gmax_kernel(z_ref, mean_ref, sq_ref, ga_ref, be_ref, o_ref):
    xn = (z_ref[0] - mean_ref[...]) / sq_ref[...]
    v = jnp.maximum(ga_ref[...] * xn + be_ref[...], 0.0)
    o_ref[0] = jnp.max(v, axis=0, keepdims=True)


def _gmax(z, mean, sq, ga, be):
    B, S, O = z.shape
    out = pl.pallas_call(
        _gmax_kernel,
        grid=(B,),
        in_specs=[
            pl.BlockSpec((1, S, O), lambda b_: (b_, 0, 0)),
            pl.BlockSpec((1, O), lambda b_: (0, 0)),
            pl.BlockSpec((1, O), lambda b_: (0, 0)),
            pl.BlockSpec((1, O), lambda b_: (0, 0)),
            pl.BlockSpec((1, O), lambda b_: (0, 0)),
        ],
        out_specs=pl.BlockSpec((1, 1, O), lambda b_: (b_, 0, 0)),
        out_shape=jax.ShapeDtypeStruct((B, 1, O), F32),
    )(z, mean.reshape(1, O), sq.reshape(1, O), ga.reshape(1, O),
      be.reshape(1, O))
    return out.reshape(B, O)


def _bn_stats(z, s1, S, count, KB):
    mean = (s1 / count).reshape(-1)
    s2c = _cvar(z, S, mean, KB)
    sq = jnp.sqrt(s2c.reshape(-1) / count + 1e-5)
    return mean, sq


def _kb_for(K, S, O):
    kb = K
    while kb > 1 and kb * S * O * 4 > 4 * 1024 * 1024:
        kb //= 2
    return kb


def _sa_layer(xyz, points, layer_params, npoint, radius, K):
    B, N, _ = xyz.shape
    new_xyz = _fps(xyz, npoint)
    idx = _ball_query(new_xyz, xyz, radius, K)
    if points is None:
        ptsfull = xyz
    else:
        ptsfull = jnp.concatenate([xyz, points], axis=2)
    C = ptsfull.shape[2]
    S = npoint
    qfull = jnp.concatenate(
        [new_xyz, jnp.zeros((B, S, C - 3), F32)], axis=2)

    (w1, b1, g1, be1), (w2, b2, g2, be2), (w3, b3, g3, be3) = layer_params
    count = float(B * K * S)
    kb1 = _kb_for(K, S, w1.shape[0])
    z, s1 = _gmlp1(ptsfull, idx, qfull, w1, b1, K, kb1)
    mean, sq = _bn_stats(z, s1, S, count, kb1)
    kb2 = _kb_for(K, S, w2.shape[0])
    z, s1 = _mlp(z, S, mean, sq, g1, be1, w2, b2, kb2)
    mean, sq = _bn_stats(z, s1, S, count, kb2)
    kb3 = _kb_for(K, S, w3.shape[0])
    z, s1 = _mlp(z, S, mean, sq, g2, be2, w3, b3, kb3)
    mean, sq = _bn_stats(z, s1, S, count, kb3)
    new_points = _maxpool(z, B, S, mean, sq, g3, be3, kb3)
    return new_xyz, new_points


def _sa_group_all(xyz, points, layer_params):
    B, N, _ = xyz.shape
    ptsfull = jnp.concatenate([xyz, points], axis=2)
    (w1, b1, g1, be1), (w2, b2, g2, be2), (w3, b3, g3, be3) = layer_params
    count = float(B * N)
    z, s1 = _lin(ptsfull, w1, b1)
    mean, sq = _bn_stats(z, s1, N, count, 1)
    z, s1 = _mlp(z, N, mean, sq, g1, be1, w2, b2, 1)
    mean, sq = _bn_stats(z, s1, N, count, 1)
    z, s1 = _mlp(z, N, mean, sq, g2, be2, w3, b3, 1)
    mean, sq = _bn_stats(z, s1, N, count, 1)
    return _gmax(z, mean, sq, g3, be3)


def kernel(xyz, params):
    B = xyz.shape[0]
    pts = jnp.transpose(xyz, (0, 2, 1))
    xyz3 = pts[:, :, :3]
    norm = pts[:, :, 3:]
    l1_xyz, l1_points = _sa_layer(xyz3, norm, params['sa1'], 778, 0.2, 16)
    l2_xyz, l2_points = _sa_layer(l1_xyz, l1_points, params['sa2'], 388, 0.4, 32)
    l3_xyz, l3_points = _sa_layer(l2_xyz, l2_points, params['sa3'], 194, 0.8, 64)
    out = _sa_group_all(l3_xyz, l3_points, params['sa4'])
    return out.reshape(B, 512)


# speedup vs baseline: 9.4767x; 1.0001x over previous
import functools

import jax
import jax.numpy as jnp
from jax import lax
from jax.experimental import pallas as pl
from jax.experimental.pallas import tpu as pltpu
from jax.experimental.pallas import tpu_sc as plsc

F32 = jnp.float32


def _twosum(s, c, v):
    t = s + v
    e = jnp.where(jnp.abs(s) >= jnp.abs(v), (s - t) + v, (v - t) + s)
    return t, c + e


def _fps_kernel(xs_ref, ys_ref, zs_ref, nx_ref, ny_ref, nz_ref, *, npoint):
    X = xs_ref[...]
    Y = ys_ref[...]
    Z = zs_ref[...]
    B, N = X.shape
    iota_n = lax.broadcasted_iota(jnp.int32, (B, N), 1)
    iota_s = lax.broadcasted_iota(jnp.int32, (B, npoint), 1)

    def body(i, state):
        dist, far, nx, ny, nz = state
        oh = (iota_n == far).astype(F32)
        cx = jnp.sum(X * oh, axis=1, keepdims=True)
        cy = jnp.sum(Y * oh, axis=1, keepdims=True)
        cz = jnp.sum(Z * oh, axis=1, keepdims=True)
        nx = jnp.where(iota_s == i, cx, nx)
        ny = jnp.where(iota_s == i, cy, ny)
        nz = jnp.where(iota_s == i, cz, nz)
        d = (X - cx) ** 2 + (Y - cy) ** 2 + (Z - cz) ** 2
        dist = jnp.minimum(dist, d)
        m = jnp.max(dist, axis=1, keepdims=True)
        far = jnp.min(jnp.where(dist == m, iota_n, N), axis=1, keepdims=True)
        return dist, far, nx, ny, nz

    dist0 = jnp.full((B, N), 1e10, F32)
    far0 = jnp.zeros((B, 1), jnp.int32)
    z0 = jnp.zeros((B, npoint), F32)
    _, _, nx, ny, nz = lax.fori_loop(0, npoint, body, (dist0, far0, z0, z0, z0))
    nx_ref[...] = nx
    ny_ref[...] = ny
    nz_ref[...] = nz


def _fps(xyz, npoint):
    B, N, _ = xyz.shape
    xs, ys, zs = xyz[:, :, 0], xyz[:, :, 1], xyz[:, :, 2]
    out = jax.ShapeDtypeStruct((B, npoint), F32)
    nx, ny, nz = pl.pallas_call(
        functools.partial(_fps_kernel, npoint=npoint),
        grid=(1,),
        in_specs=[pl.BlockSpec((B, N), lambda i: (0, 0))] * 3,
        out_specs=[pl.BlockSpec((B, npoint), lambda i: (0, 0))] * 3,
        out_shape=[out, out, out],
    )(xs, ys, zs)
    return jnp.stack([nx, ny, nz], axis=2)


def _bq_kernel(q_ref, pt_ref, idx_ref, *, radius2, K):
    q = q_ref[0]
    pt = pt_ref[0]
    S = q.shape[0]
    N = pt.shape[1]
    qn = jnp.sum(q * q, axis=1, keepdims=True)
    pn = jnp.sum(pt * pt, axis=0, keepdims=True)
    cross = lax.dot_general(q, pt, (((1,), (0,)), ((), ())))
    d = qn + pn - 2.0 * cross
    iota_n = lax.broadcasted_iota(jnp.int32, (S, N), 1)
    big = jnp.int32(N)
    key = jnp.where(d <= radius2, iota_n, big)
    iota_k = lax.broadcasted_iota(jnp.int32, (S, K), 1)
    acc = jnp.zeros((S, K), jnp.int32)
    first = None
    for j in range(K):
        mj = jnp.min(key, axis=1, keepdims=True)
        key = jnp.where(key == mj, big, key)
        if j == 0:
            first = mj
            mjf = mj
        else:
            mjf = jnp.where(mj >= big, first, mj)
        acc = jnp.where(iota_k == j, mjf, acc)
    idx_ref[0] = acc


def _ball_query(new_xyz, xyz, radius, K):
    B, S, _ = new_xyz.shape
    N = xyz.shape[1]
    xyz_t = jnp.transpose(xyz, (0, 2, 1))
    return pl.pallas_call(
        functools.partial(_bq_kernel, radius2=radius * radius, K=K),
        grid=(B,),
        in_specs=[
            pl.BlockSpec((1, S, 3), lambda b: (b, 0, 0)),
            pl.BlockSpec((1, 3, N), lambda b: (b, 0, 0)),
        ],
        out_specs=pl.BlockSpec((1, S, K), lambda b: (b, 0, 0)),
        out_shape=jax.ShapeDtypeStruct((B, S, K), jnp.int32),
    )(new_xyz, xyz_t)


def _sc_gather(table, idx_flat):
    R = idx_flat.shape[0]
    Cp = table.shape[1]
    info = plsc.get_sparse_core_info()
    NW = info.num_cores * info.num_subcores
    CH = 256
    n_chunks = R // CH
    rounds = -(-n_chunks // NW)
    mesh = plsc.VectorSubcoreMesh(core_axis_name="c", subcore_axis_name="s")

    @functools.partial(
        pl.kernel, mesh=mesh,
        out_type=jax.ShapeDtypeStruct((R, Cp), F32),
        scratch_types=[
            pltpu.VMEM((CH,), jnp.int32),
            pltpu.VMEM((CH, Cp), F32),
            pltpu.SemaphoreType.DMA,
        ],
    )
    def k(table_hbm, idx_hbm, out_hbm, idx_v, rows_v, sem):
        wid = lax.axis_index("s") * info.num_cores + lax.axis_index("c")
        for i in range(rounds):
            ck = wid + i * NW

            @pl.when(ck < n_chunks)
            def _():
                base = ck * CH
                pltpu.sync_copy(idx_hbm.at[pl.ds(base, CH)], idx_v)
                pltpu.async_copy(table_hbm.at[idx_v], rows_v, sem).wait()
                pltpu.sync_copy(rows_v, out_hbm.at[pl.ds(base, CH)])

    return k(table, idx_flat)


def _gmlp1_kernel(g_ref, qf_ref, w_ref, b_ref,
                  z_ref, s1_ref, c1_ref, *, KB):
    qf = qf_ref[0]
    w = w_ref[...]
    b = b_ref[...]
    kb = pl.program_id(1)

    @pl.when(jnp.logical_and(pl.program_id(0) == 0, kb == 0))
    def _():
        s1_ref[...] = jnp.zeros_like(s1_ref)
        c1_ref[...] = jnp.zeros_like(c1_ref)

    s1 = s1_ref[...]
    c1 = c1_ref[...]
    for k in range(KB):
        gn = g_ref[k] - qf
        zk = lax.dot_general(gn, w, (((1,), (1,)), ((), ()))) + b
        z_ref[k] = zk
        s1, c1 = _twosum(s1, c1, jnp.sum(zk, axis=0, keepdims=True))

    s1_ref[...] = s1
    c1_ref[...] = c1


def _gmlp1(ptsfull, idx, qfull, w, b, K, KB):
    B, N, C = ptsfull.shape
    S = idx.shape[1]
    O = w.shape[0]
    Cp = -(-C // 128) * 128
    pad = Cp - C
    table = jnp.concatenate(
        [ptsfull, jnp.zeros((B, N, pad), F32)], axis=2).reshape(B * N, Cp)
    qfp = jnp.concatenate([qfull, jnp.zeros((B, S, pad), F32)], axis=2)
    wp = jnp.concatenate([w, jnp.zeros((O, pad), F32)], axis=1)
    offs = (jnp.arange(B, dtype=jnp.int32) * N)[:, None, None]
    idx_flat = (jnp.transpose(idx, (0, 2, 1)) + offs).reshape(-1)
    g = _sc_gather(table, idx_flat).reshape(B * K, S, Cp)

    nkb = K // KB
    z, s1, c1 = pl.pallas_call(
        functools.partial(_gmlp1_kernel, KB=KB),
        grid=(B, nkb),
        in_specs=[
            pl.BlockSpec((KB, S, Cp), lambda b_, kb: (b_ * nkb + kb, 0, 0)),
            pl.BlockSpec((1, S, Cp), lambda b_, kb: (b_, 0, 0)),
            pl.BlockSpec((O, Cp), lambda b_, kb: (0, 0)),
            pl.BlockSpec((1, O), lambda b_, kb: (0, 0)),
        ],
        out_specs=[
            pl.BlockSpec((KB, S, O), lambda b_, kb: (b_ * nkb + kb, 0, 0)),
            pl.BlockSpec((1, O), lambda b_, kb: (0, 0)),
            pl.BlockSpec((1, O), lambda b_, kb: (0, 0)),
        ],
        out_shape=[
            jax.ShapeDtypeStruct((B * K, S, O), F32),
            jax.ShapeDtypeStruct((1, O), F32),
            jax.ShapeDtypeStruct((1, O), F32),
        ],
    )(g, qfp, wp, b.reshape(1, O))
    return z, s1 + c1


def _mlp_kernel(z_ref, mean_ref, sq_ref, ga_ref, be_ref, w_ref, b_ref,
                o_ref, s1_ref, c1_ref, *, KB):
    mean = mean_ref[...]
    sq = sq_ref[...]
    ga = ga_ref[...]
    be = be_ref[...]
    w = w_ref[...]
    b = b_ref[...]
    @pl.when(pl.program_id(0) == 0)
    def _():
        s1_ref[...] = jnp.zeros_like(s1_ref)
        c1_ref[...] = jnp.zeros_like(c1_ref)

    s1 = s1_ref[...]
    c1 = c1_ref[...]
    for k in range(KB):
        xn = (z_ref[k] - mean) / sq
        x = jnp.maximum(ga * xn + be, 0.0)
        zk = lax.dot_general(x, w, (((1,), (1,)), ((), ()))) + b
        o_ref[k] = zk
        s1, c1 = _twosum(s1, c1, jnp.sum(zk, axis=0, keepdims=True))

    s1_ref[...] = s1
    c1_ref[...] = c1


def _mlp(z, S, mean, sq, ga, be, w, b, KB):
    BK = z.shape[0]
    Oin = z.shape[2]
    O = w.shape[0]
    steps = BK // KB
    out, s1, c1 = pl.pallas_call(
        functools.partial(_mlp_kernel, KB=KB),
        grid=(steps,),
        in_specs=[
            pl.BlockSpec((KB, S, Oin), lambda i: (i, 0, 0)),
            pl.BlockSpec((1, Oin), lambda i: (0, 0)),
            pl.BlockSpec((1, Oin), lambda i: (0, 0)),
            pl.BlockSpec((1, Oin), lambda i: (0, 0)),
            pl.BlockSpec((1, Oin), lambda i: (0, 0)),
            pl.BlockSpec((O, Oin), lambda i: (0, 0)),
            pl.BlockSpec((1, O), lambda i: (0, 0)),
        ],
        out_specs=[
            pl.BlockSpec((KB, S, O), lambda i: (i, 0, 0)),
            pl.BlockSpec((1, O), lambda i: (0, 0)),
            pl.BlockSpec((1, O), lambda i: (0, 0)),
        ],
        out_shape=[
            jax.ShapeDtypeStruct((BK, S, O), F32),
            jax.ShapeDtypeStruct((1, O), F32),
            jax.ShapeDtypeStruct((1, O), F32),
        ],
    )(z, mean.reshape(1, Oin), sq.reshape(1, Oin), ga.reshape(1, Oin),
      be.reshape(1, Oin), w, b.reshape(1, O))
    return out, s1 + c1


def _maxpool_kernel(z_ref, mean_ref, sq_ref, ga_ref, be_ref, o_ref, *, KB):
    mean = mean_ref[...]
    sq = sq_ref[...]
    ga = ga_ref[...]
    be = be_ref[...]
    xn = (z_ref[0] - mean) / sq
    m = jnp.maximum(ga * xn + be, 0.0)
    for k in range(1, KB):
        xn = (z_ref[k] - mean) / sq
        m = jnp.maximum(m, jnp.maximum(ga * xn + be, 0.0))

    kb = pl.program_id(1)

    @pl.when(kb == 0)
    def _():
        o_ref[0] = m

    @pl.when(kb != 0)
    def _():
        o_ref[0] = jnp.maximum(o_ref[0], m)


def _maxpool(z, B, S, mean, sq, ga, be, KB):
    BK = z.shape[0]
    O = z.shape[2]
    K = BK // B
    nkb = K // KB
    return pl.pallas_call(
        functools.partial(_maxpool_kernel, KB=KB),
        grid=(B, nkb),
        in_specs=[
            pl.BlockSpec((KB, S, O), lambda b_, kb: (b_ * nkb + kb, 0, 0)),
            pl.BlockSpec((1, O), lambda b_, kb: (0, 0)),
            pl.BlockSpec((1, O), lambda b_, kb: (0, 0)),
            pl.BlockSpec((1, O), lambda b_, kb: (0, 0)),
            pl.BlockSpec((1, O), lambda b_, kb: (0, 0)),
        ],
        out_specs=pl.BlockSpec((1, S, O), lambda b_, kb: (b_, 0, 0)),
        out_shape=jax.ShapeDtypeStruct((B, S, O), F32),
    )(z, mean.reshape(1, O), sq.reshape(1, O), ga.reshape(1, O),
      be.reshape(1, O))


def _lin_kernel(x_ref, w_ref, b_ref, o_ref, s1_ref, c1_ref):
    x = x_ref[0]
    b = b_ref[...]
    z = lax.dot_general(x, w_ref[...], (((1,), (1,)), ((), ()))) + b
    o_ref[0] = z

    @pl.when(pl.program_id(0) == 0)
    def _():
        s1_ref[...] = jnp.zeros_like(s1_ref)
        c1_ref[...] = jnp.zeros_like(c1_ref)

    s1, c1 = _twosum(s1_ref[...], c1_ref[...],
                     jnp.sum(z, axis=0, keepdims=True))
    s1_ref[...] = s1
    c1_ref[...] = c1


def _lin(x, w, b):
    B, S, C = x.shape
    O = w.shape[0]
    z, s1, c1 = pl.pallas_call(
        _lin_kernel,
        grid=(B,),
        in_specs=[
            pl.BlockSpec((1, S, C), lambda b_: (b_, 0, 0)),
            pl.BlockSpec((O, C), lambda b_: (0, 0)),
            pl.BlockSpec((1, O), lambda b_: (0, 0)),
        ],
        out_specs=[
            pl.BlockSpec((1, S, O), lambda b_: (b_, 0, 0)),
            pl.BlockSpec((1, O), lambda b_: (0, 0)),
            pl.BlockSpec((1, O), lambda b_: (0, 0)),
        ],
        out_shape=[
            jax.ShapeDtypeStruct((B, S, O), F32),
            jax.ShapeDtypeStruct((1, O), F32),
            jax.ShapeDtypeStruct((1, O), F32),
        ],
    )(x, w, b.reshape(1, O))
    return z, s1 + c1


def _cvar_kernel(z_ref, mean_ref, s2_ref, c2_ref, *, KB):
    mean = mean_ref[...]

    @pl.when(pl.program_id(0) == 0)
    def _():
        s2_ref[...] = jnp.zeros_like(s2_ref)
        c2_ref[...] = jnp.zeros_like(c2_ref)

    s2 = s2_ref[...]
    c2 = c2_ref[...]
    for k in range(KB):
        d = z_ref[k] - mean
        s2, c2 = _twosum(s2, c2, jnp.sum(d * d, axis=0, keepdims=True))

    s2_ref[...] = s2
    c2_ref[...] = c2


def _cvar(z, S, mean, KB):
    BK = z.shape[0]
    O = z.shape[2]
    steps = BK // KB
    s2, c2 = pl.pallas_call(
        functools.partial(_cvar_kernel, KB=KB),
        grid=(steps,),
        in_specs=[
            pl.BlockSpec((KB, S, O), lambda i: (i, 0, 0)),
            pl.BlockSpec((1, O), lambda i: (0, 0)),
        ],
        out_specs=[
            pl.BlockSpec((1, O), lambda i: (0, 0)),
            pl.BlockSpec((1, O), lambda i: (0, 0)),
        ],
        out_shape=[
            jax.ShapeDtypeStruct((1, O), F32),
            jax.ShapeDtypeStruct((1, O), F32),
        ],
    )(z, mean.reshape(1, O))
    return s2 + c2


def _gmax_kernel(z_ref, mean_ref, sq_ref, ga_ref, be_ref, o_ref):
    xn = (z_ref[0] - mean_ref[...]) / sq_ref[...]
    v = jnp.maximum(ga_ref[...] * xn + be_ref[...], 0.0)
    o_ref[0] = jnp.max(v, axis=0, keepdims=True)


def _gmax(z, mean, sq, ga, be):
    B, S, O = z.shape
    out = pl.pallas_call(
        _gmax_kernel,
        grid=(B,),
        in_specs=[
            pl.BlockSpec((1, S, O), lambda b_: (b_, 0, 0)),
            pl.BlockSpec((1, O), lambda b_: (0, 0)),
            pl.BlockSpec((1, O), lambda b_: (0, 0)),
            pl.BlockSpec((1, O), lambda b_: (0, 0)),
            pl.BlockSpec((1, O), lambda b_: (0, 0)),
        ],
        out_specs=pl.BlockSpec((1, 1, O), lambda b_: (b_, 0, 0)),
        out_shape=jax.ShapeDtypeStruct((B, 1, O), F32),
    )(z, mean.reshape(1, O), sq.reshape(1, O), ga.reshape(1, O),
      be.reshape(1, O))
    return out.reshape(B, O)


def _bn_stats(z, s1, S, count, KB):
    mean = (s1 / count).reshape(-1)
    s2c = _cvar(z, S, mean, KB)
    sq = jnp.sqrt(s2c.reshape(-1) / count + 1e-5)
    return mean, sq


def _kb_for(K, S, O):
    kb = K
    while kb > 1 and kb * S * O * 4 > 4 * 1024 * 1024:
        kb //= 2
    return kb


def _sa_layer(xyz, points, layer_params, npoint, radius, K):
    B, N, _ = xyz.shape
    new_xyz = _fps(xyz, npoint)
    idx = _ball_query(new_xyz, xyz, radius, K)
    if points is None:
        ptsfull = xyz
    else:
        ptsfull = jnp.concatenate([xyz, points], axis=2)
    C = ptsfull.shape[2]
    S = npoint
    qfull = jnp.concatenate(
        [new_xyz, jnp.zeros((B, S, C - 3), F32)], axis=2)

    (w1, b1, g1, be1), (w2, b2, g2, be2), (w3, b3, g3, be3) = layer_params
    count = float(B * K * S)
    kb1 = _kb_for(K, S, w1.shape[0])
    z, s1 = _gmlp1(ptsfull, idx, qfull, w1, b1, K, kb1)
    mean, sq = _bn_stats(z, s1, S, count, kb1)
    kb2 = _kb_for(K, S, w2.shape[0])
    z, s1 = _mlp(z, S, mean, sq, g1, be1, w2, b2, kb2)
    mean, sq = _bn_stats(z, s1, S, count, kb2)
    kb3 = _kb_for(K, S, w3.shape[0])
    z, s1 = _mlp(z, S, mean, sq, g2, be2, w3, b3, kb3)
    mean, sq = _bn_stats(z, s1, S, count, kb3)
    new_points = _maxpool(z, B, S, mean, sq, g3, be3, kb3)
    return new_xyz, new_points


def _sa_group_all(xyz, points, layer_params):
    B, N, _ = xyz.shape
    ptsfull = jnp.concatenate([xyz, points], axis=2)
    (w1, b1, g1, be1), (w2, b2, g2, be2), (w3, b3, g3, be3) = layer_params
    count = float(B * N)
    z, s1 = _lin(ptsfull, w1, b1)
    mean, sq = _bn_stats(z, s1, N, count, 1)
    z, s1 = _mlp(z, N, mean, sq, g1, be1, w2, b2, 1)
    mean, sq = _bn_stats(z, s1, N, count, 1)
    z, s1 = _mlp(z, N, mean, sq, g2, be2, w3, b3, 1)
    mean, sq = _bn_stats(z, s1, N, count, 1)
    return _gmax(z, mean, sq, g3, be3)


def kernel(xyz, params):
    B = xyz.shape[0]
    pts = jnp.transpose(xyz, (0, 2, 1))
    xyz3 = pts[:, :, :3]
    norm = pts[:, :, 3:]
    l1_xyz, l1_points = _sa_layer(xyz3, norm, params['sa1'], 778, 0.2, 16)
    l2_xyz, l2_points = _sa_layer(l1_xyz, l1_points, params['sa2'], 388, 0.4, 32)
    l3_xyz, l3_points = _sa_layer(l2_xyz, l2_points, params['sa3'], 194, 0.8, 64)
    out = _sa_group_all(l3_xyz, l3_points, params['sa4'])
    return out.reshape(B, 512)
